# baseline (speedup 1.0000x reference)
import jax
import jax.numpy as jnp
from jax.experimental import pallas as pl
from jax.experimental.pallas import tpu as pltpu

CB = 2
NBUF = 8


def _body(x_hbm, t_hbm, o_hbm):
    B, P, D = x_hbm.shape
    nch = B // CB

    def inner(t_v, ibuf, obuf, tsem, insems, outsems):
        tcp = pltpu.make_async_copy(t_hbm, t_v, tsem)
        tcp.start()
        tcp.wait()
        for s in range(NBUF):
            pltpu.make_async_copy(
                x_hbm.at[pl.ds(s * CB, CB)],
                ibuf.at[pl.ds(s * CB, CB)],
                insems.at[s],
            ).start()
        for i in range(nch):
            s = i % NBUF
            if i >= NBUF:
                pltpu.make_async_copy(
                    obuf.at[pl.ds(s * CB, CB)],
                    o_hbm.at[pl.ds((i - NBUF) * CB, CB)],
                    outsems.at[s],
                ).wait()
            pltpu.make_async_copy(
                x_hbm.at[pl.ds(i * CB, CB)],
                ibuf.at[pl.ds(s * CB, CB)],
                insems.at[s],
            ).wait()
            obuf[pl.ds(s * CB, CB)] = ibuf[pl.ds(s * CB, CB)] + t_v[...]
            pltpu.make_async_copy(
                obuf.at[pl.ds(s * CB, CB)],
                o_hbm.at[pl.ds(i * CB, CB)],
                outsems.at[s],
            ).start()
            nxt = i + NBUF
            if nxt < nch:
                pltpu.make_async_copy(
                    x_hbm.at[pl.ds(nxt * CB, CB)],
                    ibuf.at[pl.ds(s * CB, CB)],
                    insems.at[s],
                ).start()
        for i in range(nch - NBUF, nch):
            s = i % NBUF
            pltpu.make_async_copy(
                obuf.at[pl.ds(s * CB, CB)],
                o_hbm.at[pl.ds(i * CB, CB)],
                outsems.at[s],
            ).wait()

    pl.run_scoped(
        inner,
        t_v=pltpu.VMEM((x_hbm.shape[1], x_hbm.shape[2]), jnp.float32),
        ibuf=pltpu.VMEM((NBUF * CB, x_hbm.shape[1], x_hbm.shape[2]), jnp.float32),
        obuf=pltpu.VMEM((NBUF * CB, x_hbm.shape[1], x_hbm.shape[2]), jnp.float32),
        tsem=pltpu.SemaphoreType.DMA,
        insems=pltpu.SemaphoreType.DMA((NBUF,)),
        outsems=pltpu.SemaphoreType.DMA((NBUF,)),
    )


def kernel(encoded_patches, pos_table):
    B, P, D = encoded_patches.shape
    return pl.pallas_call(
        _body,
        in_specs=[
            pl.BlockSpec(memory_space=pl.ANY),
            pl.BlockSpec(memory_space=pl.ANY),
        ],
        out_specs=pl.BlockSpec(memory_space=pl.ANY),
        out_shape=jax.ShapeDtypeStruct((B, P, D), jnp.float32),
    )(encoded_patches, pos_table)
